# baseline (speedup 1.0000x reference)
import functools

import jax
import jax.numpy as jnp
from jax import lax
from jax.experimental import pallas as pl
from jax.experimental.pallas import tpu as pltpu
from jax.experimental.pallas import tpu_sc as plsc

N = 10000
E = 160000
IN_F = 256
HID = 512
OUT_F = 128

NC = 2
NS = 16
STRIPE = 624
TAIL = N - NS * STRIPE
EB = 128
NBATCH_P = 1280
NROWS_ACC = N
NBREAL = E // EB


def _mesh():
    return plsc.VectorSubcoreMesh(core_axis_name="c", subcore_axis_name="s")


def _edge_loop2(gref, acc_sh, sidx_all, didx_all, rows, semr, sems, nbt):
    def start_rows(k, j):
        pltpu.async_copy(gref.at[sidx_all.at[k]], rows[j], semr[j])

    def wait_rows(j):
        pltpu.make_async_copy(gref.at[pl.ds(0, EB)], rows[j], semr[j]).wait()

    def start_scatter(k, j):
        pltpu.async_copy(rows[j], acc_sh.at[didx_all.at[k]], sems[j], add=True)

    def wait_scatter(j):
        pltpu.make_async_copy(rows[j], acc_sh.at[pl.ds(0, EB)], sems[j]).wait()

    start_rows(0, 0)
    wait_rows(0)
    start_scatter(0, 0)
    start_rows(1, 1)

    def outer(g, carry):
        for j in (1, 0):
            k = 2 * g + (1 if j == 1 else 2)
            wait_rows(j)
            wait_scatter(1 - j)
            start_rows(k + 1, 1 - j)
            start_scatter(k, j)
        return carry

    lax.fori_loop(0, (nbt - 2) // 2, outer, 0)
    wait_rows(1)
    start_scatter(nbt - 1, 1)
    wait_scatter(0)
    wait_scatter(1)


def _striped_copy(s, src, dst):
    pltpu.sync_copy(src.at[pl.ds(s * STRIPE, STRIPE)],
                    dst.at[pl.ds(s * STRIPE, STRIPE)])

    @pl.when(s == NS - 1)
    def _tail():
        pltpu.sync_copy(src.at[pl.ds(NS * STRIPE, TAIL)],
                        dst.at[pl.ds(NS * STRIPE, TAIL)])


def _deg_body(dst_hbm, ones_hbm, zeros_hbm, out_hbm,
              didx_all, ones_v, sem0, sem1, acc_sh):
    c = lax.axis_index("c")
    s = lax.axis_index("s")
    w = c * NS + s
    nbt = NBATCH_P // (NC * NS)
    pltpu.sync_copy(zeros_hbm, acc_sh.at[pl.ds(s * STRIPE, STRIPE)])

    @pl.when(s == NS - 1)
    def _tail():
        pltpu.sync_copy(zeros_hbm.at[pl.ds(0, TAIL)],
                        acc_sh.at[pl.ds(NS * STRIPE, TAIL)])

    pltpu.sync_copy(ones_hbm, ones_v)
    pltpu.sync_copy(dst_hbm.at[pl.ds(w * nbt, nbt)], didx_all)
    plsc.subcore_barrier()

    sems = (sem0, sem1)

    def start_scatter(k, j):
        pltpu.async_copy(ones_v, acc_sh.at[didx_all.at[k]], sems[j], add=True)

    def wait_scatter(j):
        pltpu.make_async_copy(ones_v, acc_sh.at[pl.ds(0, EB)], sems[j]).wait()

    nreal = jnp.clip(NBREAL - w * nbt, 0, nbt)
    start_scatter(0, 0)
    start_scatter(1, 1)

    def body(g, carry):
        for j in (0, 1):
            k = 2 * g + 2 + j
            wait_scatter(j)
            start_scatter(k, j)
        return carry

    lax.fori_loop(0, (nreal - 2) // 2, body, 0)
    wait_scatter(0)
    wait_scatter(1)
    plsc.subcore_barrier()
    _striped_copy(s, acc_sh, out_hbm.at[c])


@functools.cache
def _deg_kernel():
    return pl.kernel(
        _deg_body,
        out_type=jax.ShapeDtypeStruct((NC, N, 128), jnp.float32),
        mesh=_mesh(),
        scratch_types=[
            pltpu.VMEM((NBATCH_P // (NC * NS), EB), jnp.int32),
            pltpu.VMEM((EB, 128), jnp.float32),
            pltpu.SemaphoreType.DMA,
            pltpu.SemaphoreType.DMA,
            pltpu.VMEM_SHARED((NROWS_ACC, 128), jnp.float32),
        ],
    )


def _agg_scratch(nbt):
    return [
        pltpu.VMEM((nbt, EB), jnp.int32),
        pltpu.VMEM((nbt, EB), jnp.int32),
        pltpu.VMEM((EB, 128), jnp.float32),
        pltpu.VMEM((EB, 128), jnp.float32),
        pltpu.SemaphoreType.DMA,
        pltpu.SemaphoreType.DMA,
        pltpu.SemaphoreType.DMA,
        pltpu.SemaphoreType.DMA,
        pltpu.VMEM_SHARED((NROWS_ACC, 128), jnp.float32),
    ]


def _agg1_body(hs_hbm, src_hbm, dst_hbm, out_hbm,
               sidx_all, didx_all, rows0, rows1,
               semr0, semr1, sems0, sems1, acc_sh):
    c = lax.axis_index("c")
    s = lax.axis_index("s")
    nbt = NBATCH_P // NS

    for ch in range(4):
        @pl.when(ch % NC == c)
        def _chunk():
            hs_view = hs_hbm.at[ch]
            _striped_copy(s, hs_view, acc_sh)
            plsc.subcore_barrier()
            for half in range(2):
                base = s * nbt + half * (nbt // 2)
                pltpu.sync_copy(src_hbm.at[pl.ds(base, nbt // 2)], sidx_all)
                pltpu.sync_copy(dst_hbm.at[pl.ds(base, nbt // 2)], didx_all)
                nreal = jnp.clip(NBREAL - base, 0, nbt // 2)
                _edge_loop2(hs_view, acc_sh, sidx_all, didx_all,
                            (rows0, rows1), (semr0, semr1), (sems0, sems1),
                            nreal)
            plsc.subcore_barrier()
            _striped_copy(s, acc_sh, out_hbm.at[ch])


@functools.cache
def _agg1_kernel():
    return pl.kernel(
        _agg1_body,
        out_type=jax.ShapeDtypeStruct((4, N, 128), jnp.float32),
        mesh=_mesh(),
        scratch_types=_agg_scratch(NBATCH_P // NS // 2),
    )


def _agg2_body(hs_hbm, src_hbm, dst_hbm, out_hbm,
               sidx_all, didx_all, rows0, rows1,
               semr0, semr1, sems0, sems1, acc_sh):
    c = lax.axis_index("c")
    s = lax.axis_index("s")
    nbt = NBATCH_P // NC // NS
    base = c * (NBATCH_P // NC) + s * nbt

    _striped_copy(s, hs_hbm, acc_sh)
    pltpu.sync_copy(src_hbm.at[pl.ds(base, nbt)], sidx_all)
    pltpu.sync_copy(dst_hbm.at[pl.ds(base, nbt)], didx_all)
    plsc.subcore_barrier()
    nreal = jnp.clip(NBREAL - base, 0, nbt)
    _edge_loop2(hs_hbm, acc_sh, sidx_all, didx_all,
                (rows0, rows1), (semr0, semr1), (sems0, sems1), nreal)
    plsc.subcore_barrier()
    _striped_copy(s, acc_sh, out_hbm.at[c])


@functools.cache
def _agg2_kernel():
    return pl.kernel(
        _agg2_body,
        out_type=jax.ShapeDtypeStruct((NC, N, 128), jnp.float32),
        mesh=_mesh(),
        scratch_types=_agg_scratch(NBATCH_P // NC // NS),
    )


def _mm1_body(x_ref, w_ref, deg_ref, hs_ref, dinv_ref):
    deg = deg_ref[0, :, 0:1] + deg_ref[1, :, 0:1] + 1.0
    dinv = lax.rsqrt(jnp.maximum(deg, 1.0))
    h = jnp.dot(x_ref[...].astype(jnp.bfloat16), w_ref[...].astype(jnp.bfloat16),
                preferred_element_type=jnp.float32)
    hs_ref[0] = h * dinv
    dinv_ref[...] = dinv


def _mm1(x, w1, degw):
    rb = 1000
    grid = (N // rb, 4)
    return pl.pallas_call(
        _mm1_body,
        grid=grid,
        in_specs=[
            pl.BlockSpec((rb, IN_F), lambda i, j: (i, 0)),
            pl.BlockSpec((IN_F, 128), lambda i, j: (0, j)),
            pl.BlockSpec((NC, rb, 128), lambda i, j: (0, i, 0)),
        ],
        out_specs=[
            pl.BlockSpec((1, rb, 128), lambda i, j: (j, i, 0)),
            pl.BlockSpec((rb, 1), lambda i, j: (i, 0)),
        ],
        out_shape=[
            jax.ShapeDtypeStruct((4, N, 128), jnp.float32),
            jax.ShapeDtypeStruct((N, 1), jnp.float32),
        ],
    )(x, w1, degw)


def _mm2_body(agg_ref, dinv_ref, b1_ref, w2_ref, hs2_ref):
    dinv = dinv_ref[...]
    acc = jnp.zeros((agg_ref.shape[1], 128), jnp.float32)
    for kk in range(4):
        a = jnp.maximum(agg_ref[kk] * dinv + b1_ref[kk], 0.0)
        acc = acc + jnp.dot(a.astype(jnp.bfloat16),
                            w2_ref[kk].astype(jnp.bfloat16),
                            preferred_element_type=jnp.float32)
    hs2_ref[...] = acc * dinv


def _mm2(agg1, dinv, b1r, w2r):
    rb = 1000
    return pl.pallas_call(
        _mm2_body,
        grid=(N // rb,),
        in_specs=[
            pl.BlockSpec((4, rb, 128), lambda i: (0, i, 0)),
            pl.BlockSpec((rb, 1), lambda i: (i, 0)),
            pl.BlockSpec((4, 1, 128), lambda i: (0, 0, 0)),
            pl.BlockSpec((4, 128, 128), lambda i: (0, 0, 0)),
        ],
        out_specs=pl.BlockSpec((rb, 128), lambda i: (i, 0)),
        out_shape=jax.ShapeDtypeStruct((N, 128), jnp.float32),
    )(agg1, dinv, b1r, w2r)


def _fin_body(p_ref, hs2_ref, dinv_ref, b2_ref, out_ref):
    z = (p_ref[0] + p_ref[1] - hs2_ref[...]) * dinv_ref[...] + b2_ref[...]
    m = jnp.max(z, axis=1, keepdims=True)
    zs = z - m
    out_ref[...] = zs - jnp.log(jnp.sum(jnp.exp(zs), axis=1, keepdims=True))


def _fin(p2, hs2, dinv, b2r):
    rb = 1000
    return pl.pallas_call(
        _fin_body,
        grid=(N // rb,),
        in_specs=[
            pl.BlockSpec((NC, rb, 128), lambda i: (0, i, 0)),
            pl.BlockSpec((rb, 128), lambda i: (i, 0)),
            pl.BlockSpec((rb, 1), lambda i: (i, 0)),
            pl.BlockSpec((1, 128), lambda i: (0, 0)),
        ],
        out_specs=pl.BlockSpec((rb, 128), lambda i: (i, 0)),
        out_shape=jax.ShapeDtypeStruct((N, 128), jnp.float32),
    )(p2, hs2, dinv, b2r)


def kernel(x, edge_index, W1, b1, W2, b2):
    pad = NBATCH_P * EB - E
    spread = jnp.arange(pad, dtype=jnp.int32) % 128
    src = jnp.concatenate(
        [edge_index[0], spread]).reshape(NBATCH_P, EB)
    dst = jnp.concatenate(
        [edge_index[1], spread]).reshape(NBATCH_P, EB)
    ones16 = jnp.ones((EB, 128), jnp.float32)
    zeros16 = jnp.zeros((STRIPE, 128), jnp.float32)

    degw = _deg_kernel()(dst, ones16, zeros16)
    hs1, dinv = _mm1(x, W1, degw)
    agg1 = _agg1_kernel()(hs1, src, dst)
    hs2 = _mm2(agg1, dinv, b1.reshape(4, 1, 128), W2.reshape(4, 128, 128))
    p2 = _agg2_kernel()(hs2, src, dst)
    return _fin(p2, hs2, dinv, b2.reshape(1, 128))
